# Initial kernel scaffold; baseline (speedup 1.0000x reference)
import jax
import jax.numpy as jnp
from jax.experimental import pallas as pl


def kernel(confidence, predicted_locations, labels, gt_locations):
    raise NotImplementedError("write your pallas kernel here")


# speedup vs baseline: 1.3067x; 1.3067x over previous
import functools

import jax
import jax.numpy as jnp
from jax.experimental import pallas as pl
from jax.experimental.pallas import tpu as pltpu

_IMIN = -2147483648
_IMAX = 2147483647
_FLIP = 0x7FFFFFFF


def _stream_kernel(conf_ref, lab_ref, pred_ref, gt_ref,
                   key_ref, ce_ref, sl1_ref, glob_scr, *, C):
    b = pl.program_id(0)
    i = pl.program_id(1)
    B = pl.num_programs(0)
    nb = pl.num_programs(1)

    @pl.when(jnp.logical_and(b == 0, i == 0))
    def _init():
        glob_scr[0] = 0.0
        glob_scr[1] = 0.0

    x = conf_ref[0]
    xt = x.T
    lab = lab_ref[0, 0]
    posm = lab > 0

    m = jnp.max(xt, axis=0, keepdims=True)
    e = jnp.exp(xt - m)
    s = jnp.sum(e, axis=0, keepdims=True)
    lse = m + jnp.log(s)

    cls_iota = jax.lax.broadcasted_iota(jnp.int32, xt.shape, 0)
    xl = jnp.sum(jnp.where(cls_iota == lab, xt, 0.0), axis=0, keepdims=True)
    ce_pos = jnp.sum(jnp.where(posm, lse - xl, 0.0))

    bg = lse - xt[0:1, :]
    ib = jax.lax.bitcast_convert_type(bg, jnp.int32)
    ikey = jnp.where(ib < 0, ib ^ jnp.int32(_FLIP), ib)
    key_ref[0, 0] = jnp.where(posm, jnp.int32(_IMIN), ikey)

    p = pred_ref[0].T
    g = gt_ref[0].T
    d = p - g
    ad = jnp.abs(d)
    sl1 = jnp.where(ad < 1.0, 0.5 * d * d, ad - 0.5)
    sl1v = jnp.sum(sl1, axis=0, keepdims=True)
    sl1_part = jnp.sum(jnp.where(posm, sl1v, 0.0))

    glob_scr[0] = glob_scr[0] + ce_pos
    glob_scr[1] = glob_scr[1] + sl1_part

    @pl.when(jnp.logical_and(b == B - 1, i == nb - 1))
    def _flush():
        ce_ref[...] = jnp.reshape(glob_scr[0], (1, 1))
        sl1_ref[...] = jnp.reshape(glob_scr[1], (1, 1))


def _select_kernel(key_ref, ce_ref, sl1_ref, loc_ref, cls_ref, *, N):
    keys = key_ref[...]
    B = keys.shape[0]
    npos = jnp.sum((keys == jnp.int32(_IMIN)).astype(jnp.int32), axis=1,
                   keepdims=True)
    k = jnp.minimum(npos * 3, N - npos)
    kk = jnp.maximum(k, 1)

    def bis(_, lohi):
        lo, hi = lohi
        mid = (lo >> 1) + (hi >> 1) + (lo & hi & jnp.int32(1))
        cnt = jnp.sum((keys > mid).astype(jnp.int32), axis=1, keepdims=True)
        takes = cnt < kk
        return jnp.where(takes, lo, mid + 1), jnp.where(takes, mid, hi)

    lo, _ = jax.lax.fori_loop(
        0, 32, bis,
        (jnp.full((B, 1), _IMIN, jnp.int32), jnp.full((B, 1), _IMAX, jnp.int32)))
    t = lo
    gtm = keys > t
    cnt_gt = jnp.sum(gtm.astype(jnp.int32), axis=1, keepdims=True)
    vals = jax.lax.bitcast_convert_type(
        jnp.where(keys < 0, keys ^ jnp.int32(_FLIP), keys), jnp.float32)
    sum_gt = jnp.sum(jnp.where(gtm, vals, 0.0), axis=1, keepdims=True)
    tval = jax.lax.bitcast_convert_type(
        jnp.where(t < 0, t ^ jnp.int32(_FLIP), t), jnp.float32)
    contrib = jnp.where(k > 0,
                        sum_gt + (k - cnt_gt).astype(jnp.float32) * tval,
                        0.0)
    np_total = jnp.reshape(jnp.sum(npos).astype(jnp.float32), (1, 1))
    loc_ref[...] = sl1_ref[...] / np_total
    cls_ref[...] = (ce_ref[...] + jnp.reshape(jnp.sum(contrib), (1, 1))) / np_total


def kernel(confidence, predicted_locations, labels, gt_locations):
    B, N, C = confidence.shape
    L = predicted_locations.shape[-1]
    CHUNK = 2000 if N % 2000 == 0 else N
    nb = N // CHUNK
    labels = labels.astype(jnp.int32).reshape(B, nb, 1, CHUNK)
    keys4, ce_sum, sl1_sum = pl.pallas_call(
        functools.partial(_stream_kernel, C=C),
        grid=(B, nb),
        in_specs=[
            pl.BlockSpec((1, CHUNK, C), lambda b, i: (b, i, 0)),
            pl.BlockSpec((1, 1, 1, CHUNK), lambda b, i: (b, i, 0, 0)),
            pl.BlockSpec((1, CHUNK, L), lambda b, i: (b, i, 0)),
            pl.BlockSpec((1, CHUNK, L), lambda b, i: (b, i, 0)),
        ],
        out_specs=[
            pl.BlockSpec((1, 1, 1, CHUNK), lambda b, i: (b, i, 0, 0)),
            pl.BlockSpec((1, 1), lambda b, i: (0, 0)),
            pl.BlockSpec((1, 1), lambda b, i: (0, 0)),
        ],
        out_shape=[
            jax.ShapeDtypeStruct((B, nb, 1, CHUNK), jnp.int32),
            jax.ShapeDtypeStruct((1, 1), jnp.float32),
            jax.ShapeDtypeStruct((1, 1), jnp.float32),
        ],
        scratch_shapes=[pltpu.SMEM((2,), jnp.float32)],
        compiler_params=pltpu.CompilerParams(
            dimension_semantics=("arbitrary", "arbitrary")),
    )(confidence, labels, predicted_locations, gt_locations)

    keys = keys4.reshape(B, N)
    loc, cls = pl.pallas_call(
        functools.partial(_select_kernel, N=N),
        out_shape=[
            jax.ShapeDtypeStruct((1, 1), jnp.float32),
            jax.ShapeDtypeStruct((1, 1), jnp.float32),
        ],
    )(keys, ce_sum, sl1_sum)
    return (loc.reshape(()), cls.reshape(()))


# speedup vs baseline: 1.4845x; 1.1360x over previous
import functools

import jax
import jax.numpy as jnp
from jax.experimental import pallas as pl
from jax.experimental.pallas import tpu as pltpu

_IMIN = -2147483648
_IMAX = 2147483647
_FLIP = 0x7FFFFFFF


def _stream_kernel(conf_ref, lab_ref, pred_ref, gt_ref,
                   key_ref, ce_ref, sl1_ref, glob_scr, *, C):
    b = pl.program_id(0)
    i = pl.program_id(1)
    B = pl.num_programs(0)
    nb = pl.num_programs(1)

    @pl.when(jnp.logical_and(b == 0, i == 0))
    def _init():
        glob_scr[0] = 0.0
        glob_scr[1] = 0.0

    x = conf_ref[0]
    xt = x.T
    lab = lab_ref[0, 0]
    posm = lab > 0

    m = jnp.max(xt, axis=0, keepdims=True)
    e = jnp.exp(xt - m)
    s = jnp.sum(e, axis=0, keepdims=True)
    lse = m + jnp.log(s)

    cls_iota = jax.lax.broadcasted_iota(jnp.int32, xt.shape, 0)
    xl = jnp.sum(jnp.where(cls_iota == lab, xt, 0.0), axis=0, keepdims=True)
    ce_pos = jnp.sum(jnp.where(posm, lse - xl, 0.0))

    bg = lse - xt[0:1, :]
    ib = jax.lax.bitcast_convert_type(bg, jnp.int32)
    ikey = jnp.where(ib < 0, ib ^ jnp.int32(_FLIP), ib)
    key_ref[0, 0] = jnp.where(posm, jnp.int32(_IMIN), ikey)

    p = pred_ref[0].T
    g = gt_ref[0].T
    d = p - g
    ad = jnp.abs(d)
    sl1 = jnp.where(ad < 1.0, 0.5 * d * d, ad - 0.5)
    sl1v = jnp.sum(sl1, axis=0, keepdims=True)
    sl1_part = jnp.sum(jnp.where(posm, sl1v, 0.0))

    glob_scr[0] = glob_scr[0] + ce_pos
    glob_scr[1] = glob_scr[1] + sl1_part

    @pl.when(jnp.logical_and(b == B - 1, i == nb - 1))
    def _flush():
        ce_ref[...] = jnp.reshape(glob_scr[0], (1, 1))
        sl1_ref[...] = jnp.reshape(glob_scr[1], (1, 1))


def _select_kernel(key_ref, ce_ref, sl1_ref, loc_ref, cls_ref, *, N):
    keys = key_ref[...]
    B = keys.shape[0]
    npos = jnp.sum((keys == jnp.int32(_IMIN)).astype(jnp.int32), axis=1,
                   keepdims=True)
    k = jnp.minimum(npos * 3, N - npos)
    kk = jnp.maximum(k, 1)

    def bis(_, lohi):
        lo, hi = lohi
        mid = (lo >> 1) + (hi >> 1) + (lo & hi & jnp.int32(1))
        cnt = jnp.sum((keys > mid).astype(jnp.int32), axis=1, keepdims=True)
        takes = cnt < kk
        return jnp.where(takes, lo, mid + 1), jnp.where(takes, mid, hi)

    lo, _ = jax.lax.fori_loop(
        0, 32, bis,
        (jnp.full((B, 1), _IMIN, jnp.int32), jnp.full((B, 1), _IMAX, jnp.int32)))
    t = lo
    gtm = keys > t
    cnt_gt = jnp.sum(gtm.astype(jnp.int32), axis=1, keepdims=True)
    vals = jax.lax.bitcast_convert_type(
        jnp.where(keys < 0, keys ^ jnp.int32(_FLIP), keys), jnp.float32)
    sum_gt = jnp.sum(jnp.where(gtm, vals, 0.0), axis=1, keepdims=True)
    tval = jax.lax.bitcast_convert_type(
        jnp.where(t < 0, t ^ jnp.int32(_FLIP), t), jnp.float32)
    contrib = jnp.where(k > 0,
                        sum_gt + (k - cnt_gt).astype(jnp.float32) * tval,
                        0.0)
    np_total = jnp.reshape(jnp.sum(npos).astype(jnp.float32), (1, 1))
    loc_ref[...] = sl1_ref[...] / np_total
    cls_ref[...] = (ce_ref[...] + jnp.reshape(jnp.sum(contrib), (1, 1))) / np_total


def kernel(confidence, predicted_locations, labels, gt_locations):
    B, N, C = confidence.shape
    L = predicted_locations.shape[-1]
    CHUNK = 5000 if N % 5000 == 0 else N
    nb = N // CHUNK
    labels = labels.astype(jnp.int32).reshape(B, nb, 1, CHUNK)
    keys4, ce_sum, sl1_sum = pl.pallas_call(
        functools.partial(_stream_kernel, C=C),
        grid=(B, nb),
        in_specs=[
            pl.BlockSpec((1, CHUNK, C), lambda b, i: (b, i, 0)),
            pl.BlockSpec((1, 1, 1, CHUNK), lambda b, i: (b, i, 0, 0)),
            pl.BlockSpec((1, CHUNK, L), lambda b, i: (b, i, 0)),
            pl.BlockSpec((1, CHUNK, L), lambda b, i: (b, i, 0)),
        ],
        out_specs=[
            pl.BlockSpec((1, 1, 1, CHUNK), lambda b, i: (b, i, 0, 0)),
            pl.BlockSpec((1, 1), lambda b, i: (0, 0)),
            pl.BlockSpec((1, 1), lambda b, i: (0, 0)),
        ],
        out_shape=[
            jax.ShapeDtypeStruct((B, nb, 1, CHUNK), jnp.int32),
            jax.ShapeDtypeStruct((1, 1), jnp.float32),
            jax.ShapeDtypeStruct((1, 1), jnp.float32),
        ],
        scratch_shapes=[pltpu.SMEM((2,), jnp.float32)],
        compiler_params=pltpu.CompilerParams(
            dimension_semantics=("arbitrary", "arbitrary")),
    )(confidence, labels, predicted_locations, gt_locations)

    keys = keys4.reshape(B, N)
    loc, cls = pl.pallas_call(
        functools.partial(_select_kernel, N=N),
        out_shape=[
            jax.ShapeDtypeStruct((1, 1), jnp.float32),
            jax.ShapeDtypeStruct((1, 1), jnp.float32),
        ],
    )(keys, ce_sum, sl1_sum)
    return (loc.reshape(()), cls.reshape(()))
